# Initial kernel scaffold; baseline (speedup 1.0000x reference)
import jax
import jax.numpy as jnp
from jax.experimental import pallas as pl


def kernel(aev, W1, b1, W2, b2, W3, b3, species):
    raise NotImplementedError("write your pallas kernel here")


# speedup vs baseline: 1.7665x; 1.7665x over previous
import jax
import jax.numpy as jnp
from jax.experimental import pallas as pl

_NS = 7
_AEV = 1008
_H = 64
_CAT = _NS * _H
_BLK = 2048


def _celu(x):
    return jnp.where(x > 0, x, 0.1 * (jnp.exp(x / 0.1) - 1.0))


def _fused_body(aev_ref, sp_ref, w1_ref, b1_ref, w2_ref, b2_ref, w3_ref,
                b3_ref, out_ref):
    i = pl.program_id(0)

    aev = aev_ref[...]
    sp = sp_ref[...]
    col_sp = jax.lax.broadcasted_iota(jnp.int32, (_BLK, _CAT), 1) // _H
    mask = col_sp == sp

    h1 = jnp.dot(aev, w1_ref[...], preferred_element_type=jnp.float32)
    h1 = jnp.where(mask, _celu(h1 + b1_ref[...]), 0.0)
    hsel = h1[:, 0:_H]
    for s in range(1, _NS):
        hsel = hsel + h1[:, s * _H:(s + 1) * _H]

    h2 = jnp.dot(hsel, w2_ref[...], preferred_element_type=jnp.float32)
    h2 = jnp.where(mask, _celu(h2 + b2_ref[...]), 0.0)

    o = jnp.dot(h2, w3_ref[...], preferred_element_type=jnp.float32)

    sp7 = jax.lax.broadcasted_iota(jnp.int32, (_BLK, _NS), 1)
    b3c = jnp.where(sp7 == sp, b3_ref[...], 0.0)

    total = (jnp.sum(o, axis=(0, 1), keepdims=True)
             + jnp.sum(b3c, axis=(0, 1), keepdims=True))

    @pl.when(i == 0)
    def _():
        out_ref[...] = jnp.zeros_like(out_ref)

    out_ref[...] += total


def kernel(aev, W1, b1, W2, b2, W3, b3, species):
    n = aev.shape[0]
    w1cat = W1.transpose(1, 0, 2).reshape(_AEV, _CAT)
    b1cat = b1.reshape(1, _CAT)
    w2cat = W2.transpose(1, 0, 2).reshape(_H, _CAT)
    b2cat = b2.reshape(1, _CAT)
    w3flat = W3.reshape(_CAT, 1)
    b3row = b3.reshape(1, _NS)
    sp2d = species.reshape(n, 1)

    out = pl.pallas_call(
        _fused_body,
        grid=(n // _BLK,),
        in_specs=[
            pl.BlockSpec((_BLK, _AEV), lambda i: (i, 0)),
            pl.BlockSpec((_BLK, 1), lambda i: (i, 0)),
            pl.BlockSpec((_AEV, _CAT), lambda i: (0, 0)),
            pl.BlockSpec((1, _CAT), lambda i: (0, 0)),
            pl.BlockSpec((_H, _CAT), lambda i: (0, 0)),
            pl.BlockSpec((1, _CAT), lambda i: (0, 0)),
            pl.BlockSpec((_CAT, 1), lambda i: (0, 0)),
            pl.BlockSpec((1, _NS), lambda i: (0, 0)),
        ],
        out_specs=pl.BlockSpec((1, 1), lambda i: (0, 0)),
        out_shape=jax.ShapeDtypeStruct((1, 1), jnp.float32),
    )(aev, sp2d, w1cat, b1cat, w2cat, b2cat, w3flat, b3row)
    return out.reshape(1)
